# baseline (speedup 1.0000x reference)
import jax
import jax.numpy as jnp
from jax import lax
from jax.experimental import pallas as pl
from jax.experimental.pallas import tpu as pltpu
from jax.experimental.pallas import tpu_sc as plsc

NC = 2
NS = 16
NW = NC * NS
LANES = 16
BATCH = 16384
DIM = 32
B_PER_W = BATCH // NW
CHUNK = 128
NCHUNK = B_PER_W // CHUNK
FLAT_W = B_PER_W * DIM


def _sc_body(users_hbm, items_hbm, u_table, i_table,
             u_out, i_out, r_out,
             idx_u, idx_i, u_rows, i_rows, r_vmem, sem_g, sem_o):
    wid = lax.axis_index("s") * NC + lax.axis_index("c")
    base = wid * B_PER_W

    pltpu.sync_copy(users_hbm.at[wid], idx_u)
    pltpu.sync_copy(items_hbm.at[wid], idx_i)

    copies = []
    for j in range(NCHUNK):
        copies.append(pltpu.async_copy(
            u_table.at[idx_u.at[j]], u_rows.at[pl.ds(j * CHUNK, CHUNK)], sem_g))
        copies.append(pltpu.async_copy(
            i_table.at[idx_i.at[j]], i_rows.at[pl.ds(j * CHUNK, CHUNK)], sem_g))
    for c in copies:
        c.wait()

    out_u = pltpu.async_copy(u_rows, u_out.at[pl.ds(base, B_PER_W)], sem_o)
    out_i = pltpu.async_copy(i_rows, i_out.at[pl.ds(base, B_PER_W)], sem_o)

    row_iota = lax.iota(jnp.int32, LANES)

    def block_body(blk, carry):
        rows = blk * LANES + row_iota
        acc = jnp.zeros((LANES,), jnp.float32)
        for d in range(DIM):
            dvec = jnp.full((LANES,), d, jnp.int32)
            uv = plsc.load_gather(u_rows, [rows, dvec])
            iv = plsc.load_gather(i_rows, [rows, dvec])
            acc = acc + uv * iv
        r_vmem[pl.ds(blk * LANES, LANES)] = acc
        return carry

    lax.fori_loop(0, B_PER_W // LANES, block_body, 0)

    pltpu.sync_copy(r_vmem, r_out.at[pl.ds(base, B_PER_W)])
    out_u.wait()
    out_i.wait()


def kernel(users, items, user_embeddings, item_embeddings):
    users_r = users.astype(jnp.int32).reshape(NW, NCHUNK, CHUNK)
    items_r = items.astype(jnp.int32).reshape(NW, NCHUNK, CHUNK)
    mesh = plsc.VectorSubcoreMesh(core_axis_name="c", subcore_axis_name="s",
                                  num_cores=NC, num_subcores=NS)
    k = pl.kernel(
        _sc_body,
        out_type=(
            jax.ShapeDtypeStruct((BATCH, DIM), jnp.float32),
            jax.ShapeDtypeStruct((BATCH, DIM), jnp.float32),
            jax.ShapeDtypeStruct((BATCH,), jnp.float32),
        ),
        mesh=mesh,
        scratch_types=[
            pltpu.VMEM((NCHUNK, CHUNK), jnp.int32),
            pltpu.VMEM((NCHUNK, CHUNK), jnp.int32),
            pltpu.VMEM((B_PER_W, DIM), jnp.float32),
            pltpu.VMEM((B_PER_W, DIM), jnp.float32),
            pltpu.VMEM((B_PER_W,), jnp.float32),
            pltpu.SemaphoreType.DMA,
            pltpu.SemaphoreType.DMA,
        ],
        compiler_params=pltpu.CompilerParams(needs_layout_passes=False,
                                             use_tc_tiling_on_sc=False),
    )
    return k(users_r, items_r, user_embeddings, item_embeddings)
